# baseline (speedup 1.0000x reference)
import jax
import jax.numpy as jnp
from jax import lax
from jax.experimental import pallas as pl
from jax.experimental.pallas import tpu as pltpu

VOCAB = 100000
D = 128
CTX = 200

_BLK = 32768
_NA = (VOCAB + _BLK - 1) // _BLK
_LAST = VOCAB - (_NA - 1) * _BLK
_BLK2 = 8192
_NB = (VOCAB + _BLK2 - 1) // _BLK2
_LBUF = _NB * _BLK2


def _body(idx_ref, w1_ref, b1_ref, b2_ref, emb_hbm, w2_hbm, out_ref,
          h_ref, gbuf, buf, last_buf, lbuf, m_arr, s_arr, logz_s,
          gsem, sem, sem_last):
    i = pl.program_id(0)

    @pl.when(i == 0)
    def _():
        pltpu.make_async_copy(
            w2_hbm.at[pl.ds(0, _BLK), :], buf.at[0], sem.at[0]).start()
        for r in range(CTX):
            pltpu.make_async_copy(
                emb_hbm.at[pl.ds(idx_ref[r], 1), :],
                gbuf.at[pl.ds(r, 1), :], gsem).start()
        pltpu.make_async_copy(emb_hbm.at[pl.ds(0, CTX), :], gbuf, gsem).wait()
        e = jnp.sum(gbuf[...], axis=0, keepdims=True)
        h = jnp.dot(e, w1_ref[...].T,
                    preferred_element_type=jnp.float32) + b1_ref[...]
        h_ref[...] = jnp.maximum(h, 0.0).astype(jnp.bfloat16)

    @pl.when(i + 1 < _NA - 1)
    def _():
        slot = lax.rem(i + 1, 2)
        pltpu.make_async_copy(
            w2_hbm.at[pl.ds((i + 1) * _BLK, _BLK), :], buf.at[slot],
            sem.at[slot]).start()

    @pl.when(i + 1 == _NA - 1)
    def _():
        pltpu.make_async_copy(
            w2_hbm.at[pl.ds((_NA - 1) * _BLK, _LAST), :], last_buf,
            sem_last).start()

    def stats(k, logits):
        bm = jnp.max(logits)
        m_arr[k] = bm
        s_arr[k] = jnp.sum(jnp.exp(logits - bm))

    @pl.when(i < _NA - 1)
    def _():
        slot = lax.rem(i, 2)
        pltpu.make_async_copy(
            w2_hbm.at[pl.ds(i * _BLK, _BLK), :], buf.at[slot],
            sem.at[slot]).wait()
        logits = lax.dot_general(
            h_ref[...], buf[slot].astype(jnp.bfloat16),
            (((1,), (1,)), ((), ())), preferred_element_type=jnp.float32)
        lbuf[0:1, pl.ds(i * _BLK, _BLK)] = logits
        stats(i, logits)

    @pl.when(i == _NA - 1)
    def _():
        pltpu.make_async_copy(
            w2_hbm.at[pl.ds((_NA - 1) * _BLK, _LAST), :], last_buf,
            sem_last).wait()
        logits = lax.dot_general(
            h_ref[...], last_buf[...].astype(jnp.bfloat16),
            (((1,), (1,)), ((), ())), preferred_element_type=jnp.float32)
        lbuf[0:1, pl.ds((_NA - 1) * _BLK, _LAST)] = logits
        stats(_NA - 1, logits)

    @pl.when(i == _NA)
    def _():
        def comb(j, carry):
            m, s = carry
            mj = m_arr[j]
            mn = jnp.maximum(m, mj)
            return mn, s * jnp.exp(m - mn) + s_arr[j] * jnp.exp(mj - mn)

        m0, s0 = lax.fori_loop(1, _NA, comb, (m_arr[0], s_arr[0]))
        logz_s[0] = m0 + jnp.log(s0)

    @pl.when(i >= _NA)
    def _():
        j = i - _NA
        out_ref[...] = (lbuf[0:1, pl.ds(j * _BLK2, _BLK2)]
                        + b2_ref[...] - logz_s[0])


def _tc_main(idx, W1, b1, b2r, emb, W2):
    def b2_map(i):
        return (0, jnp.maximum(i - _NA, 0))

    return pl.pallas_call(
        _body,
        grid=(_NA + _NB,),
        in_specs=[
            pl.BlockSpec(memory_space=pltpu.SMEM),
            pl.BlockSpec((D, D), lambda i: (0, 0)),
            pl.BlockSpec((1, D), lambda i: (0, 0)),
            pl.BlockSpec((1, _BLK2), b2_map),
            pl.BlockSpec(memory_space=pl.ANY),
            pl.BlockSpec(memory_space=pl.ANY),
        ],
        out_specs=pl.BlockSpec((1, _BLK2), b2_map),
        out_shape=jax.ShapeDtypeStruct((1, VOCAB), jnp.float32),
        scratch_shapes=[
            pltpu.VMEM((1, D), jnp.bfloat16),
            pltpu.VMEM((CTX, D), jnp.float32),
            pltpu.VMEM((2, _BLK, D), jnp.float32),
            pltpu.VMEM((_LAST, D), jnp.float32),
            pltpu.VMEM((1, _LBUF), jnp.float32),
            pltpu.SMEM((_NA,), jnp.float32),
            pltpu.SMEM((_NA,), jnp.float32),
            pltpu.SMEM((1,), jnp.float32),
            pltpu.SemaphoreType.DMA,
            pltpu.SemaphoreType.DMA((2,)),
            pltpu.SemaphoreType.DMA,
        ],
    )(idx, W1, b1, b2r, emb, W2)


def kernel(inputs, emb, W1, b1, W2, b2):
    idx = inputs.astype(jnp.int32)
    return _tc_main(idx, W1, b1.reshape(1, D), b2.reshape(1, VOCAB), emb, W2)


# speedup vs baseline: 1.0228x; 1.0228x over previous
import jax
import jax.numpy as jnp
from jax import lax
from jax.experimental import pallas as pl
from jax.experimental.pallas import tpu as pltpu

VOCAB = 100000
D = 128
CTX = 200

_BLK = 24576
_NA = (VOCAB + _BLK - 1) // _BLK
_LAST = VOCAB - (_NA - 1) * _BLK
_BLK2 = 8192
_NB = (VOCAB + _BLK2 - 1) // _BLK2
_LBUF = _NB * _BLK2


def _body(idx_ref, w1_ref, b1_ref, b2_ref, emb_hbm, w2_hbm, out_ref,
          h_ref, gbuf, buf, last_buf, lbuf, m_arr, s_arr, logz_s,
          gsem, sem, sem_last):
    i = pl.program_id(0)

    @pl.when(i == 0)
    def _():
        pltpu.make_async_copy(
            w2_hbm.at[pl.ds(0, _BLK), :], buf.at[0], sem.at[0]).start()
        for r in range(CTX):
            pltpu.make_async_copy(
                emb_hbm.at[pl.ds(idx_ref[r], 1), :],
                gbuf.at[pl.ds(r, 1), :], gsem).start()
        pltpu.make_async_copy(emb_hbm.at[pl.ds(0, CTX), :], gbuf, gsem).wait()
        e = jnp.sum(gbuf[...], axis=0, keepdims=True)
        h = jnp.dot(e, w1_ref[...].T,
                    preferred_element_type=jnp.float32) + b1_ref[...]
        h_ref[...] = jnp.maximum(h, 0.0).astype(jnp.bfloat16)

    @pl.when(i + 1 < _NA - 1)
    def _():
        slot = lax.rem(i + 1, 2)
        pltpu.make_async_copy(
            w2_hbm.at[pl.ds((i + 1) * _BLK, _BLK), :], buf.at[slot],
            sem.at[slot]).start()

    @pl.when(i + 1 == _NA - 1)
    def _():
        pltpu.make_async_copy(
            w2_hbm.at[pl.ds((_NA - 1) * _BLK, _LAST), :], last_buf,
            sem_last).start()

    def stats(k, logits):
        bm = jnp.max(logits)
        m_arr[k] = bm
        s_arr[k] = jnp.sum(jnp.exp(logits - bm))

    @pl.when(i < _NA - 1)
    def _():
        slot = lax.rem(i, 2)
        pltpu.make_async_copy(
            w2_hbm.at[pl.ds(i * _BLK, _BLK), :], buf.at[slot],
            sem.at[slot]).wait()
        logits = lax.dot_general(
            h_ref[...], buf[slot].astype(jnp.bfloat16),
            (((1,), (1,)), ((), ())), preferred_element_type=jnp.float32)
        lbuf[0:1, pl.ds(i * _BLK, _BLK)] = logits
        stats(i, logits)

    @pl.when(i == _NA - 1)
    def _():
        pltpu.make_async_copy(
            w2_hbm.at[pl.ds((_NA - 1) * _BLK, _LAST), :], last_buf,
            sem_last).wait()
        logits = lax.dot_general(
            h_ref[...], last_buf[...].astype(jnp.bfloat16),
            (((1,), (1,)), ((), ())), preferred_element_type=jnp.float32)
        lbuf[0:1, pl.ds((_NA - 1) * _BLK, _LAST)] = logits
        stats(_NA - 1, logits)

    @pl.when(i == _NA)
    def _():
        def comb(j, carry):
            m, s = carry
            mj = m_arr[j]
            mn = jnp.maximum(m, mj)
            return mn, s * jnp.exp(m - mn) + s_arr[j] * jnp.exp(mj - mn)

        m0, s0 = lax.fori_loop(1, _NA, comb, (m_arr[0], s_arr[0]))
        logz_s[0] = m0 + jnp.log(s0)

    @pl.when(i >= _NA)
    def _():
        j = i - _NA
        out_ref[...] = (lbuf[0:1, pl.ds(j * _BLK2, _BLK2)]
                        + b2_ref[...] - logz_s[0])


def _tc_main(idx, W1, b1, b2r, emb, W2):
    def b2_map(i):
        return (0, jnp.maximum(i - _NA, 0))

    return pl.pallas_call(
        _body,
        grid=(_NA + _NB,),
        in_specs=[
            pl.BlockSpec(memory_space=pltpu.SMEM),
            pl.BlockSpec((D, D), lambda i: (0, 0)),
            pl.BlockSpec((1, D), lambda i: (0, 0)),
            pl.BlockSpec((1, _BLK2), b2_map),
            pl.BlockSpec(memory_space=pl.ANY),
            pl.BlockSpec(memory_space=pl.ANY),
        ],
        out_specs=pl.BlockSpec((1, _BLK2), b2_map),
        out_shape=jax.ShapeDtypeStruct((1, VOCAB), jnp.float32),
        scratch_shapes=[
            pltpu.VMEM((1, D), jnp.bfloat16),
            pltpu.VMEM((CTX, D), jnp.float32),
            pltpu.VMEM((2, _BLK, D), jnp.float32),
            pltpu.VMEM((_LAST, D), jnp.float32),
            pltpu.VMEM((1, _LBUF), jnp.float32),
            pltpu.SMEM((_NA,), jnp.float32),
            pltpu.SMEM((_NA,), jnp.float32),
            pltpu.SMEM((1,), jnp.float32),
            pltpu.SemaphoreType.DMA,
            pltpu.SemaphoreType.DMA((2,)),
            pltpu.SemaphoreType.DMA,
        ],
    )(idx, W1, b1, b2r, emb, W2)


def kernel(inputs, emb, W1, b1, W2, b2):
    idx = inputs.astype(jnp.int32)
    return _tc_main(idx, W1, b1.reshape(1, D), b2.reshape(1, VOCAB), emb, W2)


# speedup vs baseline: 1.2147x; 1.1875x over previous
import jax
import jax.numpy as jnp
from jax import lax
from jax.experimental import pallas as pl
from jax.experimental.pallas import tpu as pltpu

VOCAB = 100000
D = 128
CTX = 200

_BLK = 16384
_NA = (VOCAB + _BLK - 1) // _BLK
_LAST = VOCAB - (_NA - 1) * _BLK
_LBUF = _NA * _BLK


def _body(idx_ref, w1_ref, b1_ref, b2_ref, emb_hbm, w2_hbm, out_ref,
          h_ref, gbuf, buf_a, buf_b, last_buf, lbuf, m_arr, s_arr,
          gsem, sem_a, sem_b, sem_last):
    i = pl.program_id(0)

    def start(j, ref, sem):
        pltpu.make_async_copy(
            w2_hbm.at[pl.ds(j * _BLK, _BLK), :], ref, sem).start()

    @pl.when(i == 0)
    def _():
        start(0, buf_a, sem_a)
        start(1, buf_b, sem_b)
        for r in range(CTX):
            pltpu.make_async_copy(
                emb_hbm.at[pl.ds(idx_ref[r], 1), :],
                gbuf.at[pl.ds(r, 1), :], gsem).start()
        pltpu.make_async_copy(emb_hbm.at[pl.ds(0, CTX), :], gbuf, gsem).wait()
        e = jnp.sum(gbuf[...], axis=0, keepdims=True)
        h = jnp.dot(e, w1_ref[...].T,
                    preferred_element_type=jnp.float32) + b1_ref[...]
        h_ref[...] = jnp.maximum(h, 0.0)

    @pl.when(jnp.logical_and(i >= 1, i + 1 <= _NA - 2))
    def _():
        @pl.when(lax.rem(i + 1, 2) == 0)
        def _():
            start(i + 1, buf_a, sem_a)

        @pl.when(lax.rem(i + 1, 2) == 1)
        def _():
            start(i + 1, buf_b, sem_b)

    @pl.when(i + 1 == _NA - 1)
    def _():
        pltpu.make_async_copy(
            w2_hbm.at[pl.ds((_NA - 1) * _BLK, _LAST), :], last_buf,
            sem_last).start()

    def compute(j, ref, sem, n):
        pltpu.make_async_copy(
            w2_hbm.at[pl.ds(j * _BLK, n), :], ref, sem).wait()
        logits = lax.dot_general(
            h_ref[...], ref[...], (((1,), (1,)), ((), ())),
            preferred_element_type=jnp.float32,
            precision=lax.Precision.DEFAULT)
        lbuf[0:1, pl.ds(j * _BLK, n)] = logits
        bm = jnp.max(logits)
        m_arr[j] = bm
        s_arr[j] = jnp.sum(jnp.exp(logits - bm))

    @pl.when(jnp.logical_and(i < _NA - 1, lax.rem(i, 2) == 0))
    def _():
        compute(i, buf_a, sem_a, _BLK)

    @pl.when(jnp.logical_and(i < _NA - 1, lax.rem(i, 2) == 1))
    def _():
        compute(i, buf_b, sem_b, _BLK)

    @pl.when(i == _NA - 1)
    def _():
        compute(_NA - 1, last_buf, sem_last, _LAST)

    @pl.when(i == _NA)
    def _():
        def comb(j, carry):
            m, s = carry
            mj = m_arr[j]
            mn = jnp.maximum(m, mj)
            return mn, s * jnp.exp(m - mn) + s_arr[j] * jnp.exp(mj - mn)

        m0, s0 = lax.fori_loop(1, _NA, comb, (m_arr[0], s_arr[0]))
        logz = m0 + jnp.log(s0)
        out_ref[...] = lbuf[0:1, 0:VOCAB] + b2_ref[...] - logz


def _tc_main(idx, W1, b1, b2r, emb, W2):
    return pl.pallas_call(
        _body,
        grid=(_NA + 1,),
        in_specs=[
            pl.BlockSpec(memory_space=pltpu.SMEM),
            pl.BlockSpec((D, D), lambda i: (0, 0)),
            pl.BlockSpec((1, D), lambda i: (0, 0)),
            pl.BlockSpec((1, VOCAB), lambda i: (0, 0)),
            pl.BlockSpec(memory_space=pl.ANY),
            pl.BlockSpec(memory_space=pl.ANY),
        ],
        out_specs=pl.BlockSpec((1, VOCAB), lambda i: (0, 0)),
        out_shape=jax.ShapeDtypeStruct((1, VOCAB), jnp.float32),
        scratch_shapes=[
            pltpu.VMEM((1, D), jnp.float32),
            pltpu.VMEM((CTX, D), jnp.float32),
            pltpu.VMEM((_BLK, D), jnp.float32),
            pltpu.VMEM((_BLK, D), jnp.float32),
            pltpu.VMEM((_LAST, D), jnp.float32),
            pltpu.VMEM((1, _LBUF), jnp.float32),
            pltpu.SMEM((_NA,), jnp.float32),
            pltpu.SMEM((_NA,), jnp.float32),
            pltpu.SemaphoreType.DMA,
            pltpu.SemaphoreType.DMA,
            pltpu.SemaphoreType.DMA,
            pltpu.SemaphoreType.DMA,
        ],
    )(idx, W1, b1, b2r, emb, W2)


def kernel(inputs, emb, W1, b1, W2, b2):
    idx = inputs.astype(jnp.int32)
    return _tc_main(idx, W1, b1.reshape(1, D), b2.reshape(1, VOCAB), emb, W2)


# speedup vs baseline: 1.3280x; 1.0933x over previous
import jax
import jax.numpy as jnp
from jax import lax
from jax.experimental import pallas as pl
from jax.experimental.pallas import tpu as pltpu

VOCAB = 100000
D = 128
CTX = 200

_BLK = 16384
_NA = (VOCAB + _BLK - 1) // _BLK
_LAST = VOCAB - (_NA - 1) * _BLK
_LBUF = _NA * _BLK


def _body(idx_ref, w1_ref, b1_ref, b2_ref, emb_hbm, w2_hbm, out_ref,
          h_ref, gbuf, buf_a, buf_b, last_buf, lbuf, m_arr, s_arr,
          gsem, sem_a, sem_b, sem_last):
    i = pl.program_id(0)

    def start(j, ref, sem):
        pltpu.make_async_copy(
            w2_hbm.at[pl.ds(j * _BLK, _BLK), :], ref, sem).start()

    @pl.when(i == 0)
    def _():
        for r in range(CTX):
            pltpu.make_async_copy(
                emb_hbm.at[pl.ds(idx_ref[r], 1), :],
                gbuf.at[pl.ds(r, 1), :], gsem).start()
        start(0, buf_a, sem_a)
        start(1, buf_b, sem_b)
        pltpu.make_async_copy(emb_hbm.at[pl.ds(0, CTX), :], gbuf, gsem).wait()
        e = jnp.sum(gbuf[...], axis=0, keepdims=True)
        h = jnp.dot(e, w1_ref[...].T,
                    preferred_element_type=jnp.float32) + b1_ref[...]
        h_ref[...] = jnp.maximum(h, 0.0)

    @pl.when(jnp.logical_and(i >= 1, i + 1 <= _NA - 2))
    def _():
        @pl.when(lax.rem(i + 1, 2) == 0)
        def _():
            start(i + 1, buf_a, sem_a)

        @pl.when(lax.rem(i + 1, 2) == 1)
        def _():
            start(i + 1, buf_b, sem_b)

    @pl.when(i + 1 == _NA - 1)
    def _():
        pltpu.make_async_copy(
            w2_hbm.at[pl.ds((_NA - 1) * _BLK, _LAST), :], last_buf,
            sem_last).start()

    def compute(j, ref, sem, n):
        pltpu.make_async_copy(
            w2_hbm.at[pl.ds(j * _BLK, n), :], ref, sem).wait()
        logits = lax.dot_general(
            h_ref[...], ref[...], (((1,), (1,)), ((), ())),
            preferred_element_type=jnp.float32,
            precision=lax.Precision.DEFAULT)
        lbuf[0:1, pl.ds(j * _BLK, n)] = logits
        bm = jnp.max(logits)
        m_arr[j] = bm
        s_arr[j] = jnp.sum(jnp.exp(logits - bm))

    @pl.when(jnp.logical_and(i < _NA - 1, lax.rem(i, 2) == 0))
    def _():
        compute(i, buf_a, sem_a, _BLK)

    @pl.when(jnp.logical_and(i < _NA - 1, lax.rem(i, 2) == 1))
    def _():
        compute(i, buf_b, sem_b, _BLK)

    @pl.when(i == _NA - 1)
    def _():
        compute(_NA - 1, last_buf, sem_last, _LAST)

    @pl.when(i == _NA)
    def _():
        def comb(j, carry):
            m, s = carry
            mj = m_arr[j]
            mn = jnp.maximum(m, mj)
            return mn, s * jnp.exp(m - mn) + s_arr[j] * jnp.exp(mj - mn)

        m0, s0 = lax.fori_loop(1, _NA, comb, (m_arr[0], s_arr[0]))
        logz = m0 + jnp.log(s0)
        out_ref[...] = lbuf[0:1, 0:VOCAB] + b2_ref[...] - logz


def _tc_main(idx, W1, b1, b2r, emb, W2):
    return pl.pallas_call(
        _body,
        grid=(_NA + 1,),
        in_specs=[
            pl.BlockSpec(memory_space=pltpu.SMEM),
            pl.BlockSpec((D, D), lambda i: (0, 0)),
            pl.BlockSpec((1, D), lambda i: (0, 0)),
            pl.BlockSpec((1, VOCAB), lambda i: (0, 0)),
            pl.BlockSpec(memory_space=pl.ANY),
            pl.BlockSpec(memory_space=pl.ANY),
        ],
        out_specs=pl.BlockSpec((1, VOCAB), lambda i: (0, 0)),
        out_shape=jax.ShapeDtypeStruct((1, VOCAB), jnp.float32),
        scratch_shapes=[
            pltpu.VMEM((1, D), jnp.float32),
            pltpu.VMEM((CTX, D), jnp.float32),
            pltpu.VMEM((_BLK, D), jnp.float32),
            pltpu.VMEM((_BLK, D), jnp.float32),
            pltpu.VMEM((_LAST, D), jnp.float32),
            pltpu.VMEM((1, _LBUF), jnp.float32),
            pltpu.SMEM((_NA,), jnp.float32),
            pltpu.SMEM((_NA,), jnp.float32),
            pltpu.SemaphoreType.DMA,
            pltpu.SemaphoreType.DMA,
            pltpu.SemaphoreType.DMA,
            pltpu.SemaphoreType.DMA,
        ],
    )(idx, W1, b1, b2r, emb, W2)


def kernel(inputs, emb, W1, b1, W2, b2):
    idx = inputs.astype(jnp.int32)
    return _tc_main(idx, W1, b1.reshape(1, D), b2.reshape(1, VOCAB), emb, W2)
